# Initial kernel scaffold; baseline (speedup 1.0000x reference)
import jax
import jax.numpy as jnp
from jax.experimental import pallas as pl


def kernel(state, params):
    raise NotImplementedError("write your pallas kernel here")


# speedup vs baseline: 19.2033x; 19.2033x over previous
import jax
import jax.numpy as jnp
from jax.experimental import pallas as pl

NVIS = 32
NTOT = 36
DV = 64
DH = 128
VHEADS = 2
HHEADS = 4
KSEL = 8
TOKBLK = 64

ERF_C = [7.853861353153693e-5, -8.010193625184903e-4, 5.188327685732524e-3,
         -2.685381193529856e-2, 1.128358514861418e-1, -3.761262582423300e-1,
         1.128379165726710e+0]
ERFC_P = [2.326819970068386e-2, -1.387039388740657e-1, 3.687424674597105e-1,
          -5.824733027278666e-1, 6.210004621745983e-1, -4.944515323274145e-1,
          3.404879937665872e-1, -2.741127028184656e-1, 5.638259427386472e-1]
ERFC_R = [-1.047766399936249e+1, 1.297719955372516e+1, -7.495518717768503e+0,
          2.921019019210786e+0, -1.015265279202700e+0, 4.218463358204948e-1,
          -2.820767439740514e-1, 5.641895067754075e-1]


def _poly(w, coeffs):
    p = jnp.full_like(w, coeffs[0])
    for c in coeffs[1:]:
        p = p * w + c
    return p


def _erfc(x):
    abs_x = jnp.abs(x)
    z = jnp.exp(-x * x)
    q = 1.0 / abs_x
    y = q * q
    p = jnp.where(abs_x < 2.0, _poly(y, ERFC_P), _poly(y, ERFC_R))
    r = z * q * p
    r = jnp.where(z == 0.0, 0.0, r)
    erfc_abs = jnp.where(x < 0.0, 2.0 - r, r)
    erf_small = x * _poly(x * x, ERF_C)
    return jnp.where(abs_x < 1.0, 1.0 - erf_small, erfc_abs)


def _gelu(x):
    return 0.5 * x * _erfc(-x * (0.5 ** 0.5))


def _ln(x, g, b):
    m = jnp.mean(x, axis=-1, keepdims=True)
    v = jnp.mean((x - m) ** 2, axis=-1, keepdims=True)
    return (x - m) / jnp.sqrt(v + 1e-5) * g + b


def _dot(a, b):
    return jax.lax.dot_general(a, b, (((1,), (0,)), ((), ())))


def _row(x):
    return x.reshape(1, -1)


def _pad_col(w):
    k = w.shape[0]
    return jnp.zeros((k, 8), jnp.float32).at[:, 0].set(w)


def _flatten_params(p):
    out = []
    out.append(p["A_visible_sparse"].T)
    out.append(_row(p["r_visible_sparse"]))
    out += [p["vip1"]["W"].T, _row(p["vip1"]["b"])]
    out += [p["vip2"]["W"].T, _row(p["vip2"]["b"])]
    out.append(p["visible_node_emb"])
    for gp in p["vgat"]:
        for nm in ("q", "k", "v", "o"):
            out += [gp[nm]["W"].T, _row(gp[nm]["b"])]
    for ln in p["vnorm"]:
        out += [_row(ln["g"]), _row(ln["b"])]
    out += [p["voh1"]["W"].T, _row(p["voh1"]["b"])]
    out += [_pad_col(p["voh2"]["W"][0]), p["voh2"]["b"].reshape(1, 1)]
    out.append(p["alpha_raw"].reshape(1, 1))
    out += [_row(p["b_h2v"]), _row(p["c_h2v"])]
    out += [p["hip1"]["W"].T, _row(p["hip1"]["b"])]
    out += [p["hip2"]["W"].T, _row(p["hip2"]["b"])]
    out.append(p["hidden_node_emb"])
    for gp in p["hgat"]:
        for nm in ("q", "k", "v", "o"):
            out += [gp[nm]["W"].T, _row(gp[nm]["b"])]
    for ln in p["hnorm"]:
        out += [_row(ln["g"]), _row(ln["b"])]
    out += [p["hoh1"]["W"].T, _row(p["hoh1"]["b"])]
    out += [_pad_col(p["hoh2"]["W"][0]), p["hoh2"]["b"].reshape(1, 1)]
    out.append(_row(p["r_hidden"]))
    return out


def _gat_block(x, N, D, heads, wq, bq, wk, bk, wv, bv, wo, bo, attn_ref):
    TOK = x.shape[0]
    dh = D // heads
    flat = x.reshape(TOK * N, D)
    q = _dot(flat, wq) + bq
    k = _dot(flat, wk) + bk
    v = _dot(flat, wv) + bv
    out_heads = []
    for h in range(heads):
        sl = slice(h * dh, (h + 1) * dh)
        qh = q[:, sl].reshape(TOK, N, dh)
        kh = k[:, sl].reshape(TOK, N, dh)
        vh = v[:, sl].reshape(TOK, N, dh)
        sc = jax.lax.dot_general(
            qh, kh, (((2,), (2,)), ((0,), (0,)))) / dh ** 0.5
        work = sc
        thr = None
        for _ in range(KSEL):
            thr = jnp.max(work, axis=-1, keepdims=True)
            work = jnp.where(work >= thr, -jnp.inf, work)
        masked = jnp.where(sc >= thr, sc, -jnp.inf)
        mx = jnp.max(masked, axis=-1, keepdims=True)
        e = jnp.exp(masked - mx)
        a = e / jnp.sum(e, axis=-1, keepdims=True)
        attn_ref[:, h] = a
        oh = jax.lax.dot_general(a, vh, (((2,), (1,)), ((0,), (0,))))
        out_heads.append(oh)
    out = jnp.concatenate(out_heads, axis=-1).reshape(TOK * N, D)
    out = (_dot(out, wo) + bo).reshape(TOK, N, D)
    return out


def _fused(s_ref, *refs):
    out_full, a_v0, a_v1, a_h0, a_h1 = refs[-5:]
    it = iter(refs[:-5])

    def nxt():
        return next(it)[...]

    A_T = nxt(); r_vis = nxt()
    vip1_W = nxt(); vip1_b = nxt()
    vip2_W = nxt(); vip2_b = nxt()
    vemb = nxt()
    vgat = [tuple(nxt() for _ in range(8)) for _ in range(2)]
    vnorm = [(nxt(), nxt()) for _ in range(2)]
    voh1_W = nxt(); voh1_b = nxt(); voh2_W = nxt(); voh2_b = nxt()
    alpha_raw = nxt()
    b_h2v = nxt(); c_h2v = nxt()
    hip1_W = nxt(); hip1_b = nxt()
    hip2_W = nxt(); hip2_b = nxt()
    hemb = nxt()
    hgat = [tuple(nxt() for _ in range(8)) for _ in range(2)]
    hnorm = [(nxt(), nxt()) for _ in range(2)]
    hoh1_W = nxt(); hoh1_b = nxt(); hoh2_W = nxt(); hoh2_b = nxt()
    r_hid = nxt()

    s = s_ref[...]
    TOK = s.shape[0]

    vis = s[:, :NVIS]
    logv = jnp.log(jnp.clip(vis, 1e-6, None))
    vll = _dot(vis, A_T) + r_vis
    fv = jnp.stack([vis, logv], axis=-1).reshape(TOK * NVIS, 2)
    hv = _gelu(_dot(fv, vip1_W) + vip1_b)
    hv = (_dot(hv, vip2_W) + vip2_b).reshape(TOK, NVIS, DV) + vemb
    x = hv
    for (gw, ln), aref in zip(zip(vgat, vnorm), (a_v0, a_v1)):
        d = _gat_block(x, NVIS, DV, VHEADS, *gw, aref)
        x = _ln(x + d, ln[0], ln[1])
    g1 = _gelu(_dot(x.reshape(TOK * NVIS, DV), voh1_W) + voh1_b)
    gc = (_dot(g1, voh2_W) + voh2_b[0, 0]).reshape(TOK, NVIS, 8)[:, :, 0]
    alpha = jax.nn.sigmoid(alpha_raw[0, 0])
    base = vll + alpha * gc
    h_t = s[:, NVIS:NVIS + 1]
    coup = h_t * b_h2v + (h_t * h_t) * c_h2v
    vlr = base + coup

    logs = jnp.log(jnp.clip(s, 1e-6, None))
    fa = jnp.stack([s, logs], axis=-1).reshape(TOK * NTOT, 2)
    hh = _gelu(_dot(fa, hip1_W) + hip1_b)
    hh = (_dot(hh, hip2_W) + hip2_b).reshape(TOK, NTOT, DH) + hemb
    y = hh
    for (gw, ln), aref in zip(zip(hgat, hnorm), (a_h0, a_h1)):
        d = _gat_block(y, NTOT, DH, HHEADS, *gw, aref)
        y = _ln(y + d, ln[0], ln[1])
    g2 = _gelu(_dot(y.reshape(TOK * NTOT, DH), hoh1_W) + hoh1_b)
    hlf = (_dot(g2, hoh2_W) + hoh2_b[0, 0]).reshape(TOK, NTOT, 8)[:, :, 0]
    hlr = hlf[:, NVIS:] + r_hid

    out_full[...] = jnp.concatenate([vlr, hlr], axis=1)


def kernel(state, params):
    Bb, Tt, Nt = state.shape
    tokens = Bb * Tt
    s2 = state.reshape(tokens, Nt)
    flat = _flatten_params(params)
    grid = (tokens // TOKBLK,)
    in_specs = [pl.BlockSpec((TOKBLK, Nt), lambda i: (i, 0))]
    for a in flat:
        nd = a.ndim
        in_specs.append(
            pl.BlockSpec(a.shape, (lambda i, nd=nd: (0,) * nd)))
    out_shape = [
        jax.ShapeDtypeStruct((tokens, Nt), jnp.float32),
        jax.ShapeDtypeStruct((tokens, VHEADS, NVIS, NVIS), jnp.float32),
        jax.ShapeDtypeStruct((tokens, VHEADS, NVIS, NVIS), jnp.float32),
        jax.ShapeDtypeStruct((tokens, HHEADS, NTOT, NTOT), jnp.float32),
        jax.ShapeDtypeStruct((tokens, HHEADS, NTOT, NTOT), jnp.float32),
    ]
    out_specs = [
        pl.BlockSpec((TOKBLK, Nt), lambda i: (i, 0)),
        pl.BlockSpec((TOKBLK, VHEADS, NVIS, NVIS), lambda i: (i, 0, 0, 0)),
        pl.BlockSpec((TOKBLK, VHEADS, NVIS, NVIS), lambda i: (i, 0, 0, 0)),
        pl.BlockSpec((TOKBLK, HHEADS, NTOT, NTOT), lambda i: (i, 0, 0, 0)),
        pl.BlockSpec((TOKBLK, HHEADS, NTOT, NTOT), lambda i: (i, 0, 0, 0)),
    ]
    full, a0, a1, a2, a3 = pl.pallas_call(
        _fused,
        grid=grid,
        in_specs=in_specs,
        out_specs=out_specs,
        out_shape=out_shape,
    )(s2, *flat)
    return full.reshape(Bb, Tt, Nt), (a0, a1, a2, a3)


# speedup vs baseline: 19.3331x; 1.0068x over previous
import jax
import jax.numpy as jnp
from jax.experimental import pallas as pl

NVIS = 32
NTOT = 36
DV = 64
DH = 128
VHEADS = 2
HHEADS = 4
KSEL = 8
TOKBLK = 64

ERF_C = [7.853861353153693e-5, -8.010193625184903e-4, 5.188327685732524e-3,
         -2.685381193529856e-2, 1.128358514861418e-1, -3.761262582423300e-1,
         1.128379165726710e+0]
ERFC_P = [2.326819970068386e-2, -1.387039388740657e-1, 3.687424674597105e-1,
          -5.824733027278666e-1, 6.210004621745983e-1, -4.944515323274145e-1,
          3.404879937665872e-1, -2.741127028184656e-1, 5.638259427386472e-1]
ERFC_R = [-1.047766399936249e+1, 1.297719955372516e+1, -7.495518717768503e+0,
          2.921019019210786e+0, -1.015265279202700e+0, 4.218463358204948e-1,
          -2.820767439740514e-1, 5.641895067754075e-1]


def _poly(w, coeffs):
    p = jnp.full_like(w, coeffs[0])
    for c in coeffs[1:]:
        p = p * w + c
    return p


def _erfc(x):
    abs_x = jnp.abs(x)
    z = jnp.exp(-x * x)
    q = 1.0 / abs_x
    y = q * q
    p = jnp.where(abs_x < 2.0, _poly(y, ERFC_P), _poly(y, ERFC_R))
    r = z * q * p
    r = jnp.where(z == 0.0, 0.0, r)
    erfc_abs = jnp.where(x < 0.0, 2.0 - r, r)
    erf_small = x * _poly(x * x, ERF_C)
    return jnp.where(abs_x < 1.0, 1.0 - erf_small, erfc_abs)


def _gelu(x):
    return 0.5 * x * _erfc(-x * (0.5 ** 0.5))


def _ln(x, g, b):
    m = jnp.mean(x, axis=-1, keepdims=True)
    v = jnp.mean((x - m) ** 2, axis=-1, keepdims=True)
    return (x - m) / jnp.sqrt(v + 1e-5) * g + b


def _dot(a, b):
    return jax.lax.dot_general(a, b, (((1,), (0,)), ((), ())))


def _row(x):
    return x.reshape(1, -1)


def _pad_col(w):
    k = w.shape[0]
    return jnp.zeros((k, 8), jnp.float32).at[:, 0].set(w)


def _flatten_params(p):
    out = []
    out.append(p["A_visible_sparse"].T)
    out.append(_row(p["r_visible_sparse"]))
    out += [p["vip1"]["W"].T, _row(p["vip1"]["b"])]
    out += [p["vip2"]["W"].T, _row(p["vip2"]["b"])]
    out.append(p["visible_node_emb"])
    for gp in p["vgat"]:
        for nm in ("q", "k", "v", "o"):
            out += [gp[nm]["W"].T, _row(gp[nm]["b"])]
    for ln in p["vnorm"]:
        out += [_row(ln["g"]), _row(ln["b"])]
    out += [p["voh1"]["W"].T, _row(p["voh1"]["b"])]
    out += [_pad_col(p["voh2"]["W"][0]), p["voh2"]["b"].reshape(1, 1)]
    out.append(p["alpha_raw"].reshape(1, 1))
    out += [_row(p["b_h2v"]), _row(p["c_h2v"])]
    out += [p["hip1"]["W"].T, _row(p["hip1"]["b"])]
    out += [p["hip2"]["W"].T, _row(p["hip2"]["b"])]
    out.append(p["hidden_node_emb"])
    for gp in p["hgat"]:
        for nm in ("q", "k", "v", "o"):
            out += [gp[nm]["W"].T, _row(gp[nm]["b"])]
    for ln in p["hnorm"]:
        out += [_row(ln["g"]), _row(ln["b"])]
    out += [p["hoh1"]["W"].T, _row(p["hoh1"]["b"])]
    out += [_pad_col(p["hoh2"]["W"][0]), p["hoh2"]["b"].reshape(1, 1)]
    out.append(_row(p["r_hidden"]))
    return out


def _gat_block(x, N, D, heads, P, wq, bq, wk, bk, wv, bv, wo, bo, attn_ref):
    TOK = x.shape[0]
    G = TOK // P
    NP = N * P
    dh = D // heads
    flat = x.reshape(TOK * N, D)
    q = _dot(flat, wq) + bq
    k = _dot(flat, wk) + bk
    v = _dot(flat, wv) + bv
    ri = jax.lax.broadcasted_iota(jnp.int32, (NP, NP), 0) // N
    ci = jax.lax.broadcasted_iota(jnp.int32, (NP, NP), 1) // N
    bd = ri == ci
    out_heads = []
    for h in range(heads):
        sl = slice(h * dh, (h + 1) * dh)
        qh = q[:, sl].reshape(G, NP, dh)
        kh = k[:, sl].reshape(G, NP, dh)
        vh = v[:, sl].reshape(G, NP, dh)
        sc = jax.lax.dot_general(
            qh, kh, (((2,), (2,)), ((0,), (0,)))) / dh ** 0.5
        sc = jnp.where(bd, sc, -jnp.inf)
        work = sc
        thr = None
        for _ in range(KSEL):
            thr = jnp.max(work, axis=-1, keepdims=True)
            work = jnp.where(work >= thr, -jnp.inf, work)
        masked = jnp.where(sc >= thr, sc, -jnp.inf)
        mx = jnp.max(masked, axis=-1, keepdims=True)
        e = jnp.exp(masked - mx)
        a = e / jnp.sum(e, axis=-1, keepdims=True)
        blocks = [a[:, i * N:(i + 1) * N, i * N:(i + 1) * N] for i in range(P)]
        attn_ref[:, h] = jnp.stack(blocks, axis=1).reshape(TOK, N, N)
        oh = jax.lax.dot_general(a, vh, (((2,), (1,)), ((0,), (0,))))
        out_heads.append(oh.reshape(TOK, N, dh))
    out = jnp.concatenate(out_heads, axis=-1).reshape(TOK * N, D)
    out = (_dot(out, wo) + bo).reshape(TOK, N, D)
    return out


def _fused(s_ref, *refs):
    out_full, a_v0, a_v1, a_h0, a_h1 = refs[-5:]
    it = iter(refs[:-5])

    def nxt():
        return next(it)[...]

    A_T = nxt(); r_vis = nxt()
    vip1_W = nxt(); vip1_b = nxt()
    vip2_W = nxt(); vip2_b = nxt()
    vemb = nxt()
    vgat = [tuple(nxt() for _ in range(8)) for _ in range(2)]
    vnorm = [(nxt(), nxt()) for _ in range(2)]
    voh1_W = nxt(); voh1_b = nxt(); voh2_W = nxt(); voh2_b = nxt()
    alpha_raw = nxt()
    b_h2v = nxt(); c_h2v = nxt()
    hip1_W = nxt(); hip1_b = nxt()
    hip2_W = nxt(); hip2_b = nxt()
    hemb = nxt()
    hgat = [tuple(nxt() for _ in range(8)) for _ in range(2)]
    hnorm = [(nxt(), nxt()) for _ in range(2)]
    hoh1_W = nxt(); hoh1_b = nxt(); hoh2_W = nxt(); hoh2_b = nxt()
    r_hid = nxt()

    s = s_ref[...]
    TOK = s.shape[0]

    vis = s[:, :NVIS]
    logv = jnp.log(jnp.clip(vis, 1e-6, None))
    vll = _dot(vis, A_T) + r_vis
    fv = jnp.stack([vis, logv], axis=-1).reshape(TOK * NVIS, 2)
    hv = _gelu(_dot(fv, vip1_W) + vip1_b)
    hv = (_dot(hv, vip2_W) + vip2_b).reshape(TOK, NVIS, DV) + vemb
    x = hv
    for (gw, ln), aref in zip(zip(vgat, vnorm), (a_v0, a_v1)):
        d = _gat_block(x, NVIS, DV, VHEADS, 4, *gw, aref)
        x = _ln(x + d, ln[0], ln[1])
    g1 = _gelu(_dot(x.reshape(TOK * NVIS, DV), voh1_W) + voh1_b)
    gc = (_dot(g1, voh2_W) + voh2_b[0, 0]).reshape(TOK, NVIS, 8)[:, :, 0]
    alpha = jax.nn.sigmoid(alpha_raw[0, 0])
    base = vll + alpha * gc
    h_t = s[:, NVIS:NVIS + 1]
    coup = h_t * b_h2v + (h_t * h_t) * c_h2v
    vlr = base + coup

    logs = jnp.log(jnp.clip(s, 1e-6, None))
    fa = jnp.stack([s, logs], axis=-1).reshape(TOK * NTOT, 2)
    hh = _gelu(_dot(fa, hip1_W) + hip1_b)
    hh = (_dot(hh, hip2_W) + hip2_b).reshape(TOK, NTOT, DH) + hemb
    y = hh
    for (gw, ln), aref in zip(zip(hgat, hnorm), (a_h0, a_h1)):
        d = _gat_block(y, NTOT, DH, HHEADS, 2, *gw, aref)
        y = _ln(y + d, ln[0], ln[1])
    g2 = _gelu(_dot(y.reshape(TOK * NTOT, DH), hoh1_W) + hoh1_b)
    hlf = (_dot(g2, hoh2_W) + hoh2_b[0, 0]).reshape(TOK, NTOT, 8)[:, :, 0]
    hlr = hlf[:, NVIS:] + r_hid

    out_full[...] = jnp.concatenate([vlr, hlr], axis=1)


def kernel(state, params):
    Bb, Tt, Nt = state.shape
    tokens = Bb * Tt
    s2 = state.reshape(tokens, Nt)
    flat = _flatten_params(params)
    grid = (tokens // TOKBLK,)
    in_specs = [pl.BlockSpec((TOKBLK, Nt), lambda i: (i, 0))]
    for a in flat:
        nd = a.ndim
        in_specs.append(
            pl.BlockSpec(a.shape, (lambda i, nd=nd: (0,) * nd)))
    out_shape = [
        jax.ShapeDtypeStruct((tokens, Nt), jnp.float32),
        jax.ShapeDtypeStruct((tokens, VHEADS, NVIS, NVIS), jnp.float32),
        jax.ShapeDtypeStruct((tokens, VHEADS, NVIS, NVIS), jnp.float32),
        jax.ShapeDtypeStruct((tokens, HHEADS, NTOT, NTOT), jnp.float32),
        jax.ShapeDtypeStruct((tokens, HHEADS, NTOT, NTOT), jnp.float32),
    ]
    out_specs = [
        pl.BlockSpec((TOKBLK, Nt), lambda i: (i, 0)),
        pl.BlockSpec((TOKBLK, VHEADS, NVIS, NVIS), lambda i: (i, 0, 0, 0)),
        pl.BlockSpec((TOKBLK, VHEADS, NVIS, NVIS), lambda i: (i, 0, 0, 0)),
        pl.BlockSpec((TOKBLK, HHEADS, NTOT, NTOT), lambda i: (i, 0, 0, 0)),
        pl.BlockSpec((TOKBLK, HHEADS, NTOT, NTOT), lambda i: (i, 0, 0, 0)),
    ]
    full, a0, a1, a2, a3 = pl.pallas_call(
        _fused,
        grid=grid,
        in_specs=in_specs,
        out_specs=out_specs,
        out_shape=out_shape,
    )(s2, *flat)
    return full.reshape(Bb, Tt, Nt), (a0, a1, a2, a3)
